# Initial kernel scaffold; baseline (speedup 1.0000x reference)
import jax
import jax.numpy as jnp
from jax.experimental import pallas as pl


def kernel(x, edge_index, W1, W2, W3, attn, bias):
    raise NotImplementedError("write your pallas kernel here")


# speedup vs baseline: 12.8527x; 12.8527x over previous
import functools

import jax
import jax.numpy as jnp
from jax import lax
from jax.experimental import pallas as pl
from jax.experimental.pallas import tpu as pltpu
from jax.experimental.pallas import tpu_sc as plsc

N = 10000
E = 320000
D = 128
NEG = 0.2
NC = 2
NS = 16
NW = NC * NS
EPW = E // NW
EPT = E // NS
CHUNK = 96
NFULL = EPW // CHUNK
TAIL = EPW - NFULL * CHUNK
DFULL = EPT // CHUNK
DTAIL = EPT - DFULL * CHUNK
NPAD = 10240
DROWS = NPAD // D
RPT = NPAD // NS
BN = 1000


def _dense_body(x_ref, w1_ref, w2_ref, a8_ref, h_ref, at_ref):
    xb = x_ref[...]
    h_ref[...] = jnp.dot(xb, w1_ref[...], preferred_element_type=jnp.float32)
    q = lax.dot_general(a8_ref[...], w2_ref[...], (((1,), (1,)), ((), ())),
                        preferred_element_type=jnp.float32)
    at_ref[...] = lax.dot_general(q, xb, (((1,), (1,)), ((), ())),
                                  preferred_element_type=jnp.float32)


_dense = pl.pallas_call(
    _dense_body,
    out_shape=[
        jax.ShapeDtypeStruct((N, D), jnp.float32),
        jax.ShapeDtypeStruct((8, N), jnp.float32),
    ],
)


def _edge_body(src_hbm, dst_hbm, as_hbm, ad_hbm, h_hbm, out_hbm,
               acc_sh, den_sh, as_v, ad_v, src_v, dst_v, w_v, rows_v,
               den_v, denw_v, idx80_v, st16_v, dt16_v, sd32_v, dd32_v, sem):
    c = lax.axis_index("c")
    s = lax.axis_index("s")
    wid = c * NS + s
    zero16 = jnp.zeros((16,), jnp.float32)

    def zrow(i, carry):
        for k in range(D // 16):
            rows_v[i, pl.ds(k * 16, 16)] = zero16
        return carry
    lax.fori_loop(0, CHUNK, zrow, 0)

    def zden(i, carry):
        for k in range(D // 16):
            den_v[i, pl.ds(k * 16, 16)] = zero16
        return carry
    lax.fori_loop(0, DROWS, zden, 0)

    for g in range(DROWS // 16):
        idx80_v[pl.ds(g * 16, 16)] = lax.iota(jnp.int32, 16) + g * 16

    for m in range(RPT // CHUNK):
        pltpu.sync_copy(rows_v, acc_sh.at[pl.ds(s * RPT + m * CHUNK, CHUNK), :])
    pltpu.sync_copy(rows_v.at[pl.ds(0, RPT - (RPT // CHUNK) * CHUNK), :],
                    acc_sh.at[pl.ds(s * RPT + (RPT // CHUNK) * CHUNK,
                                    RPT - (RPT // CHUNK) * CHUNK), :])

    @pl.when(s == 0)
    def _():
        pltpu.sync_copy(rows_v.at[pl.ds(0, DROWS), :], den_sh)

    pltpu.sync_copy(as_hbm, as_v)
    pltpu.sync_copy(ad_hbm, ad_v)

    plsc.subcore_barrier()

    def wgroup(sv, dv):
        e = plsc.load_gather(as_v, [sv]) + plsc.load_gather(ad_v, [dv])
        e = jnp.where(e >= 0, e, NEG * e)
        return jnp.exp(e)

    def den_edges(n_edges, src_ref, dst_ref):
        for q in range(n_edges // 16):
            sv = src_ref[pl.ds(q * 16, 16)]
            dv = dst_ref[pl.ds(q * 16, 16)]
            w = wgroup(sv, dv)
            plsc.addupdate_scatter(
                den_v,
                [lax.shift_right_logical(dv, 7), jnp.bitwise_and(dv, 127)],
                w)

    based = s * EPT

    def den_chunk(j, carry):
        off = based + j * CHUNK
        pltpu.sync_copy(src_hbm.at[pl.ds(off, CHUNK)], src_v)
        pltpu.sync_copy(dst_hbm.at[pl.ds(off, CHUNK)], dst_v)
        den_edges(CHUNK, src_v, dst_v)
        return carry
    lax.fori_loop(0, DFULL, den_chunk, 0)

    offdt = based + DFULL * CHUNK
    pltpu.sync_copy(src_hbm.at[pl.ds(offdt, DTAIL)], sd32_v)
    pltpu.sync_copy(dst_hbm.at[pl.ds(offdt, DTAIL)], dd32_v)
    den_edges(DTAIL, sd32_v, dd32_v)

    pltpu.sync_copy(den_v, den_sh.at[idx80_v], add=True)
    plsc.subcore_barrier()
    dwin = jnp.minimum((5 * s) & ~7, DROWS - 16)
    dwin = pl.multiple_of(dwin, 8)
    pltpu.sync_copy(den_sh.at[pl.ds(dwin, 16), :], denw_v)

    def msg_edges(n_edges, src_ref, dst_ref):
        for q in range(n_edges // 16):
            sv = src_ref[pl.ds(q * 16, 16)]
            dv = dst_ref[pl.ds(q * 16, 16)]
            w_v[pl.ds(q * 16, 16)] = wgroup(sv, dv)
        pltpu.async_copy(h_hbm.at[src_ref], rows_v.at[pl.ds(0, n_edges), :],
                         sem).wait()
        def srow(i, carry):
            wi = plsc.load_gather(w_v, [jnp.full((16,), i, jnp.int32)])
            for k in range(D // 16):
                rows_v[i, pl.ds(k * 16, 16)] = \
                    rows_v[i, pl.ds(k * 16, 16)] * wi
            return carry
        lax.fori_loop(0, n_edges, srow, 0)
        pltpu.sync_copy(rows_v.at[pl.ds(0, n_edges), :], acc_sh.at[dst_ref],
                        add=True)

    base = wid * EPW

    def msg_chunk(j, carry):
        off = base + j * CHUNK
        pltpu.sync_copy(src_hbm.at[pl.ds(off, CHUNK)], src_v)
        pltpu.sync_copy(dst_hbm.at[pl.ds(off, CHUNK)], dst_v)
        msg_edges(CHUNK, src_v, dst_v)
        return carry
    lax.fori_loop(0, NFULL, msg_chunk, 0)

    offt = base + NFULL * CHUNK
    pltpu.sync_copy(src_hbm.at[pl.ds(offt, TAIL)], st16_v)
    pltpu.sync_copy(dst_hbm.at[pl.ds(offt, TAIL)], dt16_v)
    msg_edges(TAIL, st16_v, dt16_v)

    plsc.subcore_barrier()
    ncopy = -(-RPT // CHUNK)
    for m in range(ncopy):
        rows = min(CHUNK, RPT - m * CHUNK)
        row0 = s * RPT + m * CHUNK
        pltpu.sync_copy(acc_sh.at[pl.ds(row0, rows), :],
                        rows_v.at[pl.ds(0, rows), :])

        def drow(r, carry):
            node_off = m * CHUNK + r
            drow_i = 5 * s + lax.shift_right_logical(node_off, 7) - dwin
            dcol_i = jnp.bitwise_and(node_off, 127)
            dsp = plsc.load_gather(
                denw_v,
                [jnp.full((16,), drow_i, jnp.int32),
                 jnp.full((16,), dcol_i, jnp.int32)])
            rinv = 1.0 / jnp.where(dsp > 0, dsp, 1.0)
            for k in range(D // 16):
                rows_v[r, pl.ds(k * 16, 16)] = \
                    rows_v[r, pl.ds(k * 16, 16)] * rinv
            return carry
        lax.fori_loop(0, rows, drow, 0)
        pltpu.sync_copy(rows_v.at[pl.ds(0, rows), :],
                        out_hbm.at[c, pl.ds(row0, rows), :])


_edges = functools.partial(
    pl.kernel,
    out_type=jax.ShapeDtypeStruct((NC, NPAD, D), jnp.float32),
    mesh=plsc.VectorSubcoreMesh(core_axis_name="c", subcore_axis_name="s",
                                num_cores=NC, num_subcores=NS),
    compiler_params=pltpu.CompilerParams(needs_layout_passes=False),
    scratch_types=[
        pltpu.VMEM_SHARED((NPAD, D), jnp.float32),
        pltpu.VMEM_SHARED((DROWS, D), jnp.float32),
        pltpu.VMEM((NPAD,), jnp.float32),
        pltpu.VMEM((NPAD,), jnp.float32),
        pltpu.VMEM((CHUNK,), jnp.int32),
        pltpu.VMEM((CHUNK,), jnp.int32),
        pltpu.VMEM((CHUNK,), jnp.float32),
        pltpu.VMEM((CHUNK, D), jnp.float32),
        pltpu.VMEM((DROWS, D), jnp.float32),
        pltpu.VMEM((16, D), jnp.float32),
        pltpu.VMEM((DROWS,), jnp.int32),
        pltpu.VMEM((TAIL,), jnp.int32),
        pltpu.VMEM((TAIL,), jnp.int32),
        pltpu.VMEM((DTAIL,), jnp.int32),
        pltpu.VMEM((DTAIL,), jnp.int32),
        pltpu.SemaphoreType.DMA,
    ],
)(_edge_body)


def _finish_body(acc_ref, bias_ref, out_ref):
    r = acc_ref[0] + acc_ref[1] + bias_ref[...]
    out_ref[...] = jnp.where(r >= 0, r, NEG * r)


_finish = pl.pallas_call(
    _finish_body,
    grid=(N // BN,),
    in_specs=[
        pl.BlockSpec((NC, BN, D), lambda i: (0, i, 0)),
        pl.BlockSpec((D,), lambda i: (0,)),
    ],
    out_specs=pl.BlockSpec((BN, D), lambda i: (i, 0)),
    out_shape=jax.ShapeDtypeStruct((N, D), jnp.float32),
)


@jax.jit
def kernel(x, edge_index, W1, W2, W3, attn, bias):
    a8 = jnp.zeros((8, D), jnp.float32)
    a8 = a8.at[0].set(attn[:D]).at[1].set(attn[D:])
    h_trans, alphaT = _dense(x, W1, W2, a8)
    alphaP = jnp.pad(alphaT, ((0, 0), (0, NPAD - N)))
    acc = _edges(edge_index[0], edge_index[1], alphaP[0], alphaP[1], h_trans)
    return _finish(acc, bias)


# speedup vs baseline: 18.5945x; 1.4467x over previous
import functools

import jax
import jax.numpy as jnp
from jax import lax
from jax.experimental import pallas as pl
from jax.experimental.pallas import tpu as pltpu
from jax.experimental.pallas import tpu_sc as plsc

N = 10000
E = 320000
D = 128
NEG = 0.2
NC = 2
NS = 16
NW = NC * NS
EPW = E // NW
CHUNK = 96
NFULL = EPW // CHUNK
TAIL = EPW - NFULL * CHUNK
NPAD = 10240
DROWS = NPAD // D
RPT = NPAD // NS
BN = 1000


def _dense_body(x_ref, w1_ref, w2_ref, a8_ref, h_ref, at_ref):
    xb = x_ref[...]
    h_ref[...] = jnp.dot(xb, w1_ref[...], preferred_element_type=jnp.float32)
    q = lax.dot_general(a8_ref[...], w2_ref[...], (((1,), (1,)), ((), ())),
                        preferred_element_type=jnp.float32)
    at_ref[...] = lax.dot_general(q, xb, (((1,), (1,)), ((), ())),
                                  preferred_element_type=jnp.float32)


_dense = pl.pallas_call(
    _dense_body,
    out_shape=[
        jax.ShapeDtypeStruct((N, D), jnp.float32),
        jax.ShapeDtypeStruct((8, N), jnp.float32),
    ],
)


def _edge_body(src_hbm, dst_hbm, as_hbm, ad_hbm, h_hbm, acc_hbm, den_hbm,
               acc_sh, den_sh, as_v, ad_v, src_v, dst_v, w_v, rows_v,
               den_v, idx80_v, st16_v, dt16_v, sem):
    c = lax.axis_index("c")
    s = lax.axis_index("s")
    wid = c * NS + s
    zero16 = jnp.zeros((16,), jnp.float32)

    def zrow(i, carry):
        for k in range(D // 16):
            rows_v[i, pl.ds(k * 16, 16)] = zero16
        return carry
    lax.fori_loop(0, CHUNK, zrow, 0)

    def zden(i, carry):
        for k in range(D // 16):
            den_v[i, pl.ds(k * 16, 16)] = zero16
        return carry
    lax.fori_loop(0, DROWS, zden, 0)

    for g in range(DROWS // 16):
        idx80_v[pl.ds(g * 16, 16)] = lax.iota(jnp.int32, 16) + g * 16

    for m in range(RPT // CHUNK):
        pltpu.sync_copy(rows_v, acc_sh.at[pl.ds(s * RPT + m * CHUNK, CHUNK), :])
    pltpu.sync_copy(rows_v.at[pl.ds(0, RPT - (RPT // CHUNK) * CHUNK), :],
                    acc_sh.at[pl.ds(s * RPT + (RPT // CHUNK) * CHUNK,
                                    RPT - (RPT // CHUNK) * CHUNK), :])

    @pl.when(s == 0)
    def _():
        pltpu.sync_copy(rows_v.at[pl.ds(0, DROWS), :], den_sh)

    pltpu.sync_copy(as_hbm, as_v)
    pltpu.sync_copy(ad_hbm, ad_v)

    plsc.subcore_barrier()

    def wgroup(sv, dv):
        e = plsc.load_gather(as_v, [sv]) + plsc.load_gather(ad_v, [dv])
        e = jnp.where(e >= 0, e, NEG * e)
        return jnp.exp(e)

    def msg_edges(n_edges, src_ref, dst_ref):
        for q in range(n_edges // 16):
            sv = src_ref[pl.ds(q * 16, 16)]
            dv = dst_ref[pl.ds(q * 16, 16)]
            w = wgroup(sv, dv)
            w_v[pl.ds(q * 16, 16)] = w
            plsc.addupdate_scatter(
                den_v,
                [lax.shift_right_logical(dv, 7), jnp.bitwise_and(dv, 127)],
                w)
        pltpu.async_copy(h_hbm.at[src_ref], rows_v.at[pl.ds(0, n_edges), :],
                         sem).wait()
        def srow(i, carry):
            wi = plsc.load_gather(w_v, [jnp.full((16,), i, jnp.int32)])
            for k in range(D // 16):
                rows_v[i, pl.ds(k * 16, 16)] = \
                    rows_v[i, pl.ds(k * 16, 16)] * wi
            return carry
        lax.fori_loop(0, n_edges, srow, 0)
        pltpu.sync_copy(rows_v.at[pl.ds(0, n_edges), :], acc_sh.at[dst_ref],
                        add=True)

    base = wid * EPW

    def msg_chunk(j, carry):
        off = base + j * CHUNK
        pltpu.sync_copy(src_hbm.at[pl.ds(off, CHUNK)], src_v)
        pltpu.sync_copy(dst_hbm.at[pl.ds(off, CHUNK)], dst_v)
        msg_edges(CHUNK, src_v, dst_v)
        return carry
    lax.fori_loop(0, NFULL, msg_chunk, 0)

    offt = base + NFULL * CHUNK
    pltpu.sync_copy(src_hbm.at[pl.ds(offt, TAIL)], st16_v)
    pltpu.sync_copy(dst_hbm.at[pl.ds(offt, TAIL)], dt16_v)
    msg_edges(TAIL, st16_v, dt16_v)

    pltpu.sync_copy(den_v, den_sh.at[idx80_v], add=True)
    plsc.subcore_barrier()

    pltpu.sync_copy(acc_sh.at[pl.ds(s * RPT, RPT), :],
                    acc_hbm.at[c, pl.ds(s * RPT, RPT), :])

    @pl.when(s == 0)
    def _():
        pltpu.sync_copy(den_sh, den_hbm.at[c])


_edges = functools.partial(
    pl.kernel,
    out_type=[
        jax.ShapeDtypeStruct((NC, NPAD, D), jnp.float32),
        jax.ShapeDtypeStruct((NC, DROWS, D), jnp.float32),
    ],
    mesh=plsc.VectorSubcoreMesh(core_axis_name="c", subcore_axis_name="s",
                                num_cores=NC, num_subcores=NS),
    compiler_params=pltpu.CompilerParams(needs_layout_passes=False),
    scratch_types=[
        pltpu.VMEM_SHARED((NPAD, D), jnp.float32),
        pltpu.VMEM_SHARED((DROWS, D), jnp.float32),
        pltpu.VMEM((NPAD,), jnp.float32),
        pltpu.VMEM((NPAD,), jnp.float32),
        pltpu.VMEM((CHUNK,), jnp.int32),
        pltpu.VMEM((CHUNK,), jnp.int32),
        pltpu.VMEM((CHUNK,), jnp.float32),
        pltpu.VMEM((CHUNK, D), jnp.float32),
        pltpu.VMEM((DROWS, D), jnp.float32),
        pltpu.VMEM((DROWS,), jnp.int32),
        pltpu.VMEM((TAIL,), jnp.int32),
        pltpu.VMEM((TAIL,), jnp.int32),
        pltpu.SemaphoreType.DMA,
    ],
)(_edge_body)


def _finish_body(acc_ref, den_ref, bias_ref, out_ref):
    t = acc_ref[0] + acc_ref[1]
    d = den_ref[...]
    r = jnp.where(d > 0, t / jnp.where(d > 0, d, 1.0), 0.0) + bias_ref[...]
    out_ref[...] = jnp.where(r >= 0, r, NEG * r)


_finish = pl.pallas_call(
    _finish_body,
    grid=(N // BN,),
    in_specs=[
        pl.BlockSpec((NC, BN, D), lambda i: (0, i, 0)),
        pl.BlockSpec((BN, 1), lambda i: (i, 0)),
        pl.BlockSpec((D,), lambda i: (0,)),
    ],
    out_specs=pl.BlockSpec((BN, D), lambda i: (i, 0)),
    out_shape=jax.ShapeDtypeStruct((N, D), jnp.float32),
)


@jax.jit
def kernel(x, edge_index, W1, W2, W3, attn, bias):
    a8 = jnp.zeros((8, D), jnp.float32)
    a8 = a8.at[0].set(attn[:D]).at[1].set(attn[D:])
    h_trans, alphaT = _dense(x, W1, W2, a8)
    alphaP = jnp.pad(alphaT, ((0, 0), (0, NPAD - N)))
    acc, denT = _edges(edge_index[0], edge_index[1],
                       alphaP[0], alphaP[1], h_trans)
    den = (denT[0] + denT[1]).reshape(NPAD)[:N, None]
    return _finish(acc, den, bias)


# speedup vs baseline: 22.1122x; 1.1892x over previous
import functools

import jax
import jax.numpy as jnp
from jax import lax
from jax.experimental import pallas as pl
from jax.experimental.pallas import tpu as pltpu
from jax.experimental.pallas import tpu_sc as plsc

N = 10000
E = 320000
D = 128
NEG = 0.2
NC = 2
NS = 16
NW = NC * NS
EPW = E // NW
CHUNK = 64
NFULL = EPW // CHUNK
PAIRS = NFULL // 2
TAIL = EPW - NFULL * CHUNK
NPAD = 10240
DROWS = NPAD // D
RPT = NPAD // NS
BN = 1000


def _dense_body(x_ref, w1_ref, w2_ref, a8_ref, h_ref, at_ref):
    xb = x_ref[...]
    h_ref[...] = jnp.dot(xb, w1_ref[...], preferred_element_type=jnp.float32)
    q = lax.dot_general(a8_ref[...], w2_ref[...], (((1,), (1,)), ((), ())),
                        preferred_element_type=jnp.float32)
    at_ref[...] = lax.dot_general(q, xb, (((1,), (1,)), ((), ())),
                                  preferred_element_type=jnp.float32)


_dense = pl.pallas_call(
    _dense_body,
    out_shape=[
        jax.ShapeDtypeStruct((N, D), jnp.float32),
        jax.ShapeDtypeStruct((8, N), jnp.float32),
    ],
)


def _edge_body(src_hbm, dst_hbm, as_hbm, ad_hbm, h_hbm, acc_hbm, den_hbm,
               acc_sh, den_sh, as_v, ad_v, srca_v, dsta_v, srcb_v, dstb_v,
               wa_v, wb_v, rowsa_v, rowsb_v, den_v, idx80_v,
               st16_v, dt16_v, sema, semb):
    c = lax.axis_index("c")
    s = lax.axis_index("s")
    wid = c * NS + s
    zero16 = jnp.zeros((16,), jnp.float32)

    def zrow(i, carry):
        for k in range(D // 16):
            rowsa_v[i, pl.ds(k * 16, 16)] = zero16
        return carry
    lax.fori_loop(0, CHUNK, zrow, 0)

    def zden(i, carry):
        for k in range(D // 16):
            den_v[i, pl.ds(k * 16, 16)] = zero16
        return carry
    lax.fori_loop(0, DROWS, zden, 0)

    for g in range(DROWS // 16):
        idx80_v[pl.ds(g * 16, 16)] = lax.iota(jnp.int32, 16) + g * 16

    for m in range(RPT // CHUNK):
        pltpu.sync_copy(rowsa_v,
                        acc_sh.at[pl.ds(s * RPT + m * CHUNK, CHUNK), :])

    @pl.when(s == 0)
    def _():
        pltpu.sync_copy(rowsa_v, den_sh.at[pl.ds(0, CHUNK), :])
        pltpu.sync_copy(rowsa_v.at[pl.ds(0, DROWS - CHUNK), :],
                        den_sh.at[pl.ds(CHUNK, DROWS - CHUNK), :])

    pltpu.sync_copy(as_hbm, as_v)
    pltpu.sync_copy(ad_hbm, ad_v)

    plsc.subcore_barrier()

    def wgroup(sv, dv):
        e = plsc.load_gather(as_v, [sv]) + plsc.load_gather(ad_v, [dv])
        e = jnp.where(e >= 0, e, NEG * e)
        return jnp.exp(e)

    def front(off, src_ref, dst_ref, w_ref, rows_ref, sem):
        pltpu.sync_copy(src_hbm.at[pl.ds(off, CHUNK)], src_ref)
        pltpu.sync_copy(dst_hbm.at[pl.ds(off, CHUNK)], dst_ref)
        for q in range(CHUNK // 16):
            dv = dst_ref[pl.ds(q * 16, 16)]
            w = wgroup(src_ref[pl.ds(q * 16, 16)], dv)
            w_ref[pl.ds(q * 16, 16)] = w
            plsc.addupdate_scatter(
                den_v,
                [lax.shift_right_logical(dv, 7), jnp.bitwise_and(dv, 127)],
                w)
        pltpu.async_copy(h_hbm.at[src_ref], rows_ref, sem)

    def back(src_ref, dst_ref, w_ref, rows_ref, sem):
        pltpu.make_async_copy(h_hbm.at[src_ref], rows_ref, sem).wait()

        def srow(i, carry):
            wi = plsc.load_gather(w_ref, [jnp.full((16,), i, jnp.int32)])
            for k in range(D // 16):
                rows_ref[i, pl.ds(k * 16, 16)] = \
                    rows_ref[i, pl.ds(k * 16, 16)] * wi
            return carry
        lax.fori_loop(0, CHUNK, srow, 0)
        pltpu.sync_copy(rows_ref, acc_sh.at[dst_ref], add=True)

    base = wid * EPW

    def pair(j2, carry):
        offa = base + (2 * j2) * CHUNK
        front(offa, srca_v, dsta_v, wa_v, rowsa_v, sema)

        @pl.when(j2 > 0)
        def _():
            back(srcb_v, dstb_v, wb_v, rowsb_v, semb)

        front(offa + CHUNK, srcb_v, dstb_v, wb_v, rowsb_v, semb)
        back(srca_v, dsta_v, wa_v, rowsa_v, sema)
        return carry
    lax.fori_loop(0, PAIRS, pair, 0)
    back(srcb_v, dstb_v, wb_v, rowsb_v, semb)

    offt = base + NFULL * CHUNK
    pltpu.sync_copy(src_hbm.at[pl.ds(offt, TAIL)], st16_v)
    pltpu.sync_copy(dst_hbm.at[pl.ds(offt, TAIL)], dt16_v)
    dv = dt16_v[...]
    w = wgroup(st16_v[...], dv)
    wa_v[pl.ds(0, TAIL)] = w
    plsc.addupdate_scatter(
        den_v, [lax.shift_right_logical(dv, 7), jnp.bitwise_and(dv, 127)], w)
    pltpu.async_copy(h_hbm.at[st16_v], rowsa_v.at[pl.ds(0, TAIL), :],
                     sema).wait()

    def strow(i, carry):
        wi = plsc.load_gather(wa_v, [jnp.full((16,), i, jnp.int32)])
        for k in range(D // 16):
            rowsa_v[i, pl.ds(k * 16, 16)] = rowsa_v[i, pl.ds(k * 16, 16)] * wi
        return carry
    lax.fori_loop(0, TAIL, strow, 0)
    pltpu.sync_copy(rowsa_v.at[pl.ds(0, TAIL), :], acc_sh.at[dt16_v],
                    add=True)

    pltpu.sync_copy(den_v, den_sh.at[idx80_v], add=True)
    plsc.subcore_barrier()

    pltpu.sync_copy(acc_sh.at[pl.ds(s * RPT, RPT), :],
                    acc_hbm.at[c, pl.ds(s * RPT, RPT), :])

    @pl.when(s == 0)
    def _():
        pltpu.sync_copy(den_sh, den_hbm.at[c])


_edges = functools.partial(
    pl.kernel,
    out_type=[
        jax.ShapeDtypeStruct((NC, NPAD, D), jnp.float32),
        jax.ShapeDtypeStruct((NC, DROWS, D), jnp.float32),
    ],
    mesh=plsc.VectorSubcoreMesh(core_axis_name="c", subcore_axis_name="s",
                                num_cores=NC, num_subcores=NS),
    compiler_params=pltpu.CompilerParams(needs_layout_passes=False),
    scratch_types=[
        pltpu.VMEM_SHARED((NPAD, D), jnp.float32),
        pltpu.VMEM_SHARED((DROWS, D), jnp.float32),
        pltpu.VMEM((NPAD,), jnp.float32),
        pltpu.VMEM((NPAD,), jnp.float32),
        pltpu.VMEM((CHUNK,), jnp.int32),
        pltpu.VMEM((CHUNK,), jnp.int32),
        pltpu.VMEM((CHUNK,), jnp.int32),
        pltpu.VMEM((CHUNK,), jnp.int32),
        pltpu.VMEM((CHUNK,), jnp.float32),
        pltpu.VMEM((CHUNK,), jnp.float32),
        pltpu.VMEM((CHUNK, D), jnp.float32),
        pltpu.VMEM((CHUNK, D), jnp.float32),
        pltpu.VMEM((DROWS, D), jnp.float32),
        pltpu.VMEM((DROWS,), jnp.int32),
        pltpu.VMEM((TAIL,), jnp.int32),
        pltpu.VMEM((TAIL,), jnp.int32),
        pltpu.SemaphoreType.DMA,
        pltpu.SemaphoreType.DMA,
    ],
)(_edge_body)


def _finish_body(acc_ref, den_ref, bias_ref, out_ref):
    t = acc_ref[0] + acc_ref[1]
    d = den_ref[...]
    r = jnp.where(d > 0, t / jnp.where(d > 0, d, 1.0), 0.0) + bias_ref[...]
    out_ref[...] = jnp.where(r >= 0, r, NEG * r)


_finish = pl.pallas_call(
    _finish_body,
    grid=(N // BN,),
    in_specs=[
        pl.BlockSpec((NC, BN, D), lambda i: (0, i, 0)),
        pl.BlockSpec((BN, 1), lambda i: (i, 0)),
        pl.BlockSpec((D,), lambda i: (0,)),
    ],
    out_specs=pl.BlockSpec((BN, D), lambda i: (i, 0)),
    out_shape=jax.ShapeDtypeStruct((N, D), jnp.float32),
)


@jax.jit
def kernel(x, edge_index, W1, W2, W3, attn, bias):
    a8 = jnp.zeros((8, D), jnp.float32)
    a8 = a8.at[0].set(attn[:D]).at[1].set(attn[D:])
    h_trans, alphaT = _dense(x, W1, W2, a8)
    alphaP = jnp.pad(alphaT, ((0, 0), (0, NPAD - N)))
    acc, denT = _edges(edge_index[0], edge_index[1],
                       alphaP[0], alphaP[1], h_trans)
    den = (denT[0] + denT[1]).reshape(NPAD)[:N, None]
    return _finish(acc, den, bias)


# speedup vs baseline: 30.2140x; 1.3664x over previous
import functools

import jax
import jax.numpy as jnp
from jax import lax
from jax.experimental import pallas as pl
from jax.experimental.pallas import tpu as pltpu
from jax.experimental.pallas import tpu_sc as plsc

N = 10000
E = 320000
D = 128
NEG = 0.2
NC = 2
NS = 16
NW = NC * NS
EPW = E // NW
CHUNK = 32
NFULL = EPW // CHUNK
OUTER = NFULL // 4
TAIL = EPW - NFULL * CHUNK
NPAD = 10240
DROWS = NPAD // D
RPT = NPAD // NS
BN = 1000


def _dense_body(x_ref, w1_ref, w2_ref, a8_ref, h_ref, at_ref):
    xb = x_ref[...]
    h_ref[...] = jnp.dot(xb, w1_ref[...], preferred_element_type=jnp.float32)
    q = lax.dot_general(a8_ref[...], w2_ref[...], (((1,), (1,)), ((), ())),
                        preferred_element_type=jnp.float32)
    at_ref[...] = lax.dot_general(q, xb, (((1,), (1,)), ((), ())),
                                  preferred_element_type=jnp.float32)


_dense = pl.pallas_call(
    _dense_body,
    out_shape=[
        jax.ShapeDtypeStruct((N, D), jnp.float32),
        jax.ShapeDtypeStruct((8, N), jnp.float32),
    ],
)


def _edge_body(src_hbm, dst_hbm, as_hbm, ad_hbm, h_hbm, acc_hbm, den_hbm,
               acc_sh, den_sh, as_v, ad_v,
               rows0_v, rows1_v, rows2_v, rows3_v,
               src0_v, src1_v, src2_v, src3_v,
               dst0_v, dst1_v, dst2_v, dst3_v,
               sct0_v, sct1_v, sct2_v, sct3_v,
               w_v, den_v,
               semg0, semg1, semg2, semg3,
               sems0, sems1, sems2, sems3,
               semi0, semi1, semi2, semi3):
    c = lax.axis_index("c")
    s = lax.axis_index("s")
    wid = c * NS + s
    zero16 = jnp.zeros((16,), jnp.float32)
    rows = [rows0_v, rows1_v, rows2_v, rows3_v]
    srcv = [src0_v, src1_v, src2_v, src3_v]
    dstv = [dst0_v, dst1_v, dst2_v, dst3_v]
    sctv = [sct0_v, sct1_v, sct2_v, sct3_v]
    semg = [semg0, semg1, semg2, semg3]
    sems = [sems0, sems1, sems2, sems3]
    semi = [semi0, semi1, semi2, semi3]

    def zrow(i, carry):
        for k in range(D // 16):
            rows0_v[i, pl.ds(k * 16, 16)] = zero16
        return carry
    lax.fori_loop(0, CHUNK, zrow, 0)

    def zden(i, carry):
        for k in range(D // 16):
            den_v[i, pl.ds(k * 16, 16)] = zero16
        return carry
    lax.fori_loop(0, DROWS, zden, 0)

    for m in range(RPT // CHUNK):
        pltpu.sync_copy(rows0_v,
                        acc_sh.at[pl.ds(s * RPT + m * CHUNK, CHUNK), :])

    @pl.when(s == 0)
    def _():
        pltpu.sync_copy(rows0_v, den_sh.at[pl.ds(0, CHUNK), :])
        pltpu.sync_copy(rows0_v, den_sh.at[pl.ds(CHUNK, CHUNK), :])
        pltpu.sync_copy(rows0_v.at[pl.ds(0, DROWS - 2 * CHUNK), :],
                        den_sh.at[pl.ds(2 * CHUNK, DROWS - 2 * CHUNK), :])

    pltpu.sync_copy(as_hbm.at[pl.ds(0, N)], as_v)
    pltpu.sync_copy(ad_hbm.at[pl.ds(0, N)], ad_v)

    plsc.subcore_barrier()

    def wgroup(sv, dv):
        e = plsc.load_gather(as_v, [sv]) + plsc.load_gather(ad_v, [dv])
        e = jnp.where(e >= 0, e, NEG * e)
        return jnp.exp(e)

    def scale_rows(rows_ref):
        def srow(i, carry):
            wi = plsc.load_gather(w_v, [jnp.full((16,), i, jnp.int32)])
            for k in range(D // 16):
                rows_ref[i, pl.ds(k * 16, 16)] = \
                    rows_ref[i, pl.ds(k * 16, 16)] * wi
            return carry
        lax.fori_loop(0, CHUNK, srow, 0)

    base = wid * EPW

    def step(j, b):
        b2, b3 = (b + 2) % 4, (b + 3) % 4
        pltpu.make_async_copy(h_hbm.at[srcv[b]], rows[b], semg[b]).wait()
        for q in range(CHUNK // 16):
            dv = dstv[b][pl.ds(q * 16, 16)]
            w = wgroup(srcv[b][pl.ds(q * 16, 16)], dv)
            w_v[pl.ds(q * 16, 16)] = w
            plsc.addupdate_scatter(
                den_v,
                [lax.shift_right_logical(dv, 7), jnp.bitwise_and(dv, 127)],
                w)
            sctv[b][pl.ds(q * 16, 16)] = dv
        scale_rows(rows[b])
        pltpu.async_copy(rows[b], acc_sh.at[sctv[b]], sems[b], add=True)

        @pl.when(j + 3 < NFULL)
        def _():
            off3 = base + (j + 3) * CHUNK
            pltpu.async_copy(src_hbm.at[pl.ds(off3, CHUNK)], srcv[b3],
                             semi[b3])
            pltpu.async_copy(dst_hbm.at[pl.ds(off3, CHUNK)], dstv[b3],
                             semi[b3])

        @pl.when(j + 2 < NFULL)
        def _():
            @pl.when(j >= 1)
            def _():
                pltpu.make_async_copy(
                    src_hbm.at[pl.ds(0, CHUNK)], srcv[b2], semi[b2]).wait()
                pltpu.make_async_copy(
                    dst_hbm.at[pl.ds(0, CHUNK)], dstv[b2], semi[b2]).wait()

            @pl.when(j >= 2)
            def _():
                pltpu.make_async_copy(rows[b2], acc_sh.at[sctv[b2]],
                                      sems[b2]).wait()
            pltpu.async_copy(h_hbm.at[srcv[b2]], rows[b2], semg[b2])

    for jj in range(3):
        pltpu.sync_copy(src_hbm.at[pl.ds(base + jj * CHUNK, CHUNK)], srcv[jj])
        pltpu.sync_copy(dst_hbm.at[pl.ds(base + jj * CHUNK, CHUNK)], dstv[jj])
    pltpu.async_copy(h_hbm.at[srcv[0]], rows[0], semg[0])
    pltpu.async_copy(h_hbm.at[srcv[1]], rows[1], semg[1])

    def outer(jo, carry):
        for b in range(4):
            step(4 * jo + b, b)
        return carry
    lax.fori_loop(0, OUTER, outer, 0)

    for b in range(4):
        pltpu.make_async_copy(rows[b], acc_sh.at[sctv[b]], sems[b]).wait()

    offt = base + NFULL * CHUNK
    pltpu.sync_copy(src_hbm.at[pl.ds(offt, TAIL)],
                    src0_v.at[pl.ds(0, TAIL)])
    pltpu.sync_copy(dst_hbm.at[pl.ds(offt, TAIL)],
                    dst0_v.at[pl.ds(0, TAIL)])
    zi16 = jnp.zeros((16,), jnp.int32)
    src0_v[pl.ds(TAIL, 16)] = zi16
    dst0_v[pl.ds(TAIL, 16)] = zi16
    dv = dst0_v[pl.ds(0, TAIL)]
    w = wgroup(src0_v[pl.ds(0, TAIL)], dv)
    w_v[pl.ds(0, TAIL)] = w
    w_v[pl.ds(TAIL, 16)] = zero16
    plsc.addupdate_scatter(
        den_v, [lax.shift_right_logical(dv, 7), jnp.bitwise_and(dv, 127)], w)
    pltpu.async_copy(h_hbm.at[src0_v], rows0_v, semg0).wait()
    scale_rows(rows0_v)
    pltpu.sync_copy(rows0_v, acc_sh.at[dst0_v], add=True)

    i16 = lax.iota(jnp.int32, 16)
    sct0_v[pl.ds(0, 16)] = i16
    sct0_v[pl.ds(16, 16)] = i16 + 16
    sct1_v[pl.ds(0, 16)] = i16 + 32
    sct1_v[pl.ds(16, 16)] = i16 + 48
    sct2_v[pl.ds(0, 16)] = i16 + 64
    pltpu.sync_copy(den_v.at[pl.ds(0, CHUNK), :], den_sh.at[sct0_v], add=True)
    pltpu.sync_copy(den_v.at[pl.ds(CHUNK, CHUNK), :], den_sh.at[sct1_v],
                    add=True)
    pltpu.sync_copy(den_v.at[pl.ds(2 * CHUNK, 16), :],
                    den_sh.at[sct2_v.at[pl.ds(0, 16)]], add=True)
    plsc.subcore_barrier()

    pltpu.sync_copy(acc_sh.at[pl.ds(s * RPT, RPT), :],
                    acc_hbm.at[c, pl.ds(s * RPT, RPT), :])

    @pl.when(s == 0)
    def _():
        pltpu.sync_copy(den_sh, den_hbm.at[c])


_edges = functools.partial(
    pl.kernel,
    out_type=[
        jax.ShapeDtypeStruct((NC, NPAD, D), jnp.float32),
        jax.ShapeDtypeStruct((NC, DROWS, D), jnp.float32),
    ],
    mesh=plsc.VectorSubcoreMesh(core_axis_name="c", subcore_axis_name="s",
                                num_cores=NC, num_subcores=NS),
    compiler_params=pltpu.CompilerParams(needs_layout_passes=False),
    scratch_types=[
        pltpu.VMEM_SHARED((NPAD, D), jnp.float32),
        pltpu.VMEM_SHARED((DROWS, D), jnp.float32),
        pltpu.VMEM((N,), jnp.float32),
        pltpu.VMEM((N,), jnp.float32),
        pltpu.VMEM((CHUNK, D), jnp.float32),
        pltpu.VMEM((CHUNK, D), jnp.float32),
        pltpu.VMEM((CHUNK, D), jnp.float32),
        pltpu.VMEM((CHUNK, D), jnp.float32),
        pltpu.VMEM((CHUNK,), jnp.int32),
        pltpu.VMEM((CHUNK,), jnp.int32),
        pltpu.VMEM((CHUNK,), jnp.int32),
        pltpu.VMEM((CHUNK,), jnp.int32),
        pltpu.VMEM((CHUNK,), jnp.int32),
        pltpu.VMEM((CHUNK,), jnp.int32),
        pltpu.VMEM((CHUNK,), jnp.int32),
        pltpu.VMEM((CHUNK,), jnp.int32),
        pltpu.VMEM((CHUNK,), jnp.int32),
        pltpu.VMEM((CHUNK,), jnp.int32),
        pltpu.VMEM((CHUNK,), jnp.int32),
        pltpu.VMEM((CHUNK,), jnp.int32),
        pltpu.VMEM((CHUNK,), jnp.float32),
        pltpu.VMEM((DROWS, D), jnp.float32),
        pltpu.SemaphoreType.DMA,
        pltpu.SemaphoreType.DMA,
        pltpu.SemaphoreType.DMA,
        pltpu.SemaphoreType.DMA,
        pltpu.SemaphoreType.DMA,
        pltpu.SemaphoreType.DMA,
        pltpu.SemaphoreType.DMA,
        pltpu.SemaphoreType.DMA,
        pltpu.SemaphoreType.DMA,
        pltpu.SemaphoreType.DMA,
        pltpu.SemaphoreType.DMA,
        pltpu.SemaphoreType.DMA,
    ],
)(_edge_body)


def _finish_body(acc_ref, den_ref, bias_ref, out_ref):
    t = acc_ref[0] + acc_ref[1]
    d = den_ref[...]
    r = jnp.where(d > 0, t / jnp.where(d > 0, d, 1.0), 0.0) + bias_ref[...]
    out_ref[...] = jnp.where(r >= 0, r, NEG * r)


_finish = pl.pallas_call(
    _finish_body,
    grid=(N // BN,),
    in_specs=[
        pl.BlockSpec((NC, BN, D), lambda i: (0, i, 0)),
        pl.BlockSpec((BN, 1), lambda i: (i, 0)),
        pl.BlockSpec((D,), lambda i: (0,)),
    ],
    out_specs=pl.BlockSpec((BN, D), lambda i: (i, 0)),
    out_shape=jax.ShapeDtypeStruct((N, D), jnp.float32),
)


@jax.jit
def kernel(x, edge_index, W1, W2, W3, attn, bias):
    a8 = jnp.zeros((8, D), jnp.float32)
    a8 = a8.at[0].set(attn[:D]).at[1].set(attn[D:])
    h_trans, alphaT = _dense(x, W1, W2, a8)
    alphaP = jnp.pad(alphaT, ((0, 0), (0, NPAD - N)))
    acc, denT = _edges(edge_index[0], edge_index[1],
                       alphaP[0], alphaP[1], h_trans)
    den = (denT[0] + denT[1]).reshape(NPAD)[:N, None]
    return _finish(acc, den, bias)
